# baseline (speedup 1.0000x reference)
import functools

import jax
import jax.numpy as jnp
from jax import lax
from jax.experimental import pallas as pl
from jax.experimental.pallas import tpu as pltpu
from jax.experimental.pallas import tpu_sc as plsc

B, S, D = 2, 2048, 1024
DFF = 1024
E = 8
TOP_K = 2
N = B * S
A = N * TOP_K
TM = 256
G = A // TM + E
CAP = G * TM
LANES = 128

NW = 32
ALPHA = 1.702
LIMIT = 7.0


_RT = 512


def _router_body(x_ref, wg_ref, bg_ref, w_ref, i_ref):
    x = x_ref[...]
    s = lax.dot_general(x, wg_ref[...], (((1,), (1,)), ((), ())),
                        preferred_element_type=jnp.float32)
    col = lax.broadcasted_iota(jnp.int32, s.shape, 1)
    valid = col < E
    neg = jnp.float32(-jnp.inf)
    s = jnp.where(valid, s, neg)
    m = jnp.max(s, axis=1, keepdims=True)
    p = jnp.where(valid, jnp.exp(s - m), 0.0)
    p = p / jnp.sum(p, axis=1, keepdims=True)
    biased = jnp.where(valid, p + bg_ref[...], neg)
    m1 = jnp.max(biased, axis=1, keepdims=True)
    i1 = jnp.min(jnp.where(biased == m1, col, E), axis=1, keepdims=True)
    rest = jnp.where(col == i1, neg, biased)
    m2 = jnp.max(rest, axis=1, keepdims=True)
    i2 = jnp.min(jnp.where(rest == m2, col, E), axis=1, keepdims=True)
    w1 = jnp.sum(jnp.where(col == i1, p, 0.0), axis=1, keepdims=True)
    w2 = jnp.sum(jnp.where(col == i2, p, 0.0), axis=1, keepdims=True)
    w_ref[...] = jnp.where(col == 0, w1, jnp.where(col == 1, w2, 0.0))
    i_ref[...] = jnp.where(col == 0, i1, jnp.where(col == 1, i2, 0))


def _route(xf, wg_pad, bg_pad):
    w_pad, i_pad = pl.pallas_call(
        _router_body,
        grid=(N // _RT,),
        in_specs=[
            pl.BlockSpec((_RT, D), lambda m: (m, 0)),
            pl.BlockSpec((LANES, D), lambda m: (0, 0)),
            pl.BlockSpec((1, LANES), lambda m: (0, 0)),
        ],
        out_specs=[
            pl.BlockSpec((_RT, LANES), lambda m: (m, 0)),
            pl.BlockSpec((_RT, LANES), lambda m: (m, 0)),
        ],
        out_shape=[
            jax.ShapeDtypeStruct((N, LANES), jnp.float32),
            jax.ShapeDtypeStruct((N, LANES), jnp.int32),
        ],
    )(xf, wg_pad, bg_pad)
    return w_pad[:, :TOP_K], i_pad[:, :TOP_K]


_GCH = 40
_GROWS = CAP // NW
_GN = _GROWS // _GCH


@functools.cache
def _sc_gather_kernel():
    mesh = plsc.VectorSubcoreMesh(core_axis_name="c", subcore_axis_name="s")

    @functools.partial(
        pl.kernel,
        mesh=mesh,
        out_type=jax.ShapeDtypeStruct((CAP, D), jnp.float32),
        scratch_types=[
            pltpu.VMEM((_GN, _GCH), jnp.int32),
            pltpu.VMEM((_GCH, D), jnp.float32),
            pltpu.VMEM((_GCH, D), jnp.float32),
            pltpu.SemaphoreType.DMA,
            pltpu.SemaphoreType.DMA,
        ],
    )
    def gather(x_hbm, ids_hbm, out_hbm, idx_v, rows0, rows1, s0, s1):
        wid = lax.axis_index("s") * 2 + lax.axis_index("c")
        base = wid * _GROWS
        bufs, sems = (rows0, rows1), (s0, s1)
        pltpu.sync_copy(ids_hbm.at[wid], idx_v)
        cp = pltpu.async_copy(x_hbm.at[idx_v.at[0]], bufs[0], sems[0])
        for i in range(_GN):
            if i + 1 < _GN:
                nxt = pltpu.async_copy(x_hbm.at[idx_v.at[i + 1]],
                                       bufs[(i + 1) % 2], sems[(i + 1) % 2])
            cp.wait()
            pltpu.sync_copy(bufs[i % 2],
                            out_hbm.at[pl.ds(base + i * _GCH, _GCH)])
            if i + 1 < _GN:
                cp = nxt

    return gather


def _sc_gather(xfi, ids3):
    return _sc_gather_kernel()(xfi, ids3)


def _prep_body(w_ref, o_ref):
    v = w_ref[0].astype(jnp.bfloat16)
    rows = lax.broadcasted_iota(jnp.int32, (DFF, 2 * DFF), 0)
    cols = lax.broadcasted_iota(jnp.int32, (DFF, 2 * DFF), 1)
    sel_e = (cols == 2 * rows).astype(jnp.bfloat16)
    sel_o = (cols == 2 * rows + 1).astype(jnp.bfloat16)
    cdims = (((1,), (0,)), ((), ()))
    o_ref[0, :DFF] = lax.dot_general(
        sel_e, v, cdims,
        preferred_element_type=jnp.float32).astype(jnp.bfloat16)
    o_ref[0, DFF:] = lax.dot_general(
        sel_o, v, cdims,
        preferred_element_type=jnp.float32).astype(jnp.bfloat16)


def _prep_pair(w3d):
    m = w3d.shape[0]
    return pl.pallas_call(
        _prep_body,
        grid=(m,),
        in_specs=[pl.BlockSpec((1, 2 * DFF, D), lambda e: (e, 0, 0))],
        out_specs=pl.BlockSpec((1, 2 * DFF, D), lambda e: (e, 0, 0)),
        out_shape=jax.ShapeDtypeStruct((m, 2 * DFF, D), jnp.bfloat16),
    )(w3d)


def _cast_body(w_ref, o_ref):
    o_ref[...] = w_ref[...].astype(jnp.bfloat16)


def _prep_cast(w3d):
    m = w3d.shape[0]
    return pl.pallas_call(
        _cast_body,
        grid=(m,),
        in_specs=[pl.BlockSpec((1, D, DFF), lambda e: (e, 0, 0))],
        out_specs=pl.BlockSpec((1, D, DFF), lambda e: (e, 0, 0)),
        out_shape=jax.ShapeDtypeStruct((m, D, DFF), jnp.bfloat16),
    )(w3d)


def _ffn_math(x, w1u, w3u, w2):
    cdims = (((1,), (1,)), ((), ()))
    x16 = x.astype(jnp.bfloat16)
    h1 = lax.dot_general(x16, w1u, cdims, preferred_element_type=jnp.float32)
    h3 = lax.dot_general(x16, w3u, cdims, preferred_element_type=jnp.float32)
    a = h1[:, :DFF]
    b = h1[:, DFF:]
    c = h3[:, :DFF]
    d = h3[:, DFF:]
    glu = jnp.minimum(a * c, LIMIT)
    lin = jnp.clip(b * d, -LIMIT, LIMIT)
    g = glu * jax.nn.sigmoid(ALPHA * glu) * (lin + 1.0)
    return lax.dot_general(g.astype(jnp.bfloat16), w2, cdims,
                           preferred_element_type=jnp.float32)


def _grouped_body(texp_ref, valid_ref, xs_ref, w1u_ref, w3u_ref, w2_ref,
                  ws_ref, out_ref):
    @pl.when(valid_ref[pl.program_id(0)] == 1)
    def _():
        eo = _ffn_math(xs_ref[...], w1u_ref[0], w3u_ref[0], w2_ref[0])
        out_ref[...] = eo * ws_ref[0, 0, :][:, None]


def _grouped_ffn(xs, w1u, w3u, w2, wsort, texp, valid):
    def wmap(m, texp, valid):
        return (texp[m], 0, 0)

    return pl.pallas_call(
        _grouped_body,
        grid_spec=pltpu.PrefetchScalarGridSpec(
            num_scalar_prefetch=2,
            grid=(G,),
            in_specs=[
                pl.BlockSpec((TM, D), lambda m, t, v: (m, 0)),
                pl.BlockSpec((1, 2 * DFF, D), wmap),
                pl.BlockSpec((1, 2 * DFF, D), wmap),
                pl.BlockSpec((1, D, DFF), wmap),
                pl.BlockSpec((1, 1, TM), lambda m, t, v: (m, 0, 0)),
            ],
            out_specs=pl.BlockSpec((TM, D), lambda m, t, v: (m, 0)),
        ),
        out_shape=jax.ShapeDtypeStruct((CAP, D), jnp.float32),
    )(texp, valid, xs, w1u, w3u, w2, wsort)


_ST = 512


def _shared_body(x_ref, w1u_ref, w3u_ref, w2_ref, out_ref):
    out_ref[...] = _ffn_math(x_ref[...], w1u_ref[0], w3u_ref[0], w2_ref[0])


def _shared_ffn(xf, w1u, w3u, w2):
    return pl.pallas_call(
        _shared_body,
        grid=(N // _ST,),
        in_specs=[pl.BlockSpec((_ST, D), lambda m: (m, 0)),
                  pl.BlockSpec((1, 2 * DFF, D), lambda m: (0, 0, 0)),
                  pl.BlockSpec((1, 2 * DFF, D), lambda m: (0, 0, 0)),
                  pl.BlockSpec((1, D, DFF), lambda m: (0, 0, 0))],
        out_specs=pl.BlockSpec((_ST, D), lambda m: (m, 0)),
        out_shape=jax.ShapeDtypeStruct((N, D), jnp.float32),
    )(xf, w1u, w3u, w2)


_CCH = 16
_CTOK = N // NW
_CN = _CTOK // _CCH
_NV = D // 16


@functools.cache
def _sc_combine_kernel():
    mesh = plsc.VectorSubcoreMesh(core_axis_name="c", subcore_axis_name="s")

    @functools.partial(
        pl.kernel,
        mesh=mesh,
        out_type=jax.ShapeDtypeStruct((N, D), jnp.float32),
        scratch_types=[
            pltpu.VMEM((_CN, _CCH), jnp.int32),
            pltpu.VMEM((_CN, _CCH), jnp.int32),
            pltpu.VMEM((_CCH, D), jnp.float32),
            pltpu.VMEM((_CCH, D), jnp.float32),
            pltpu.VMEM((_CCH, D), jnp.float32),
            pltpu.VMEM((_CCH, D), jnp.float32),
            pltpu.VMEM((_CCH, D), jnp.float32),
            pltpu.VMEM((_CCH, D), jnp.float32),
            pltpu.SemaphoreType.DMA,
            pltpu.SemaphoreType.DMA,
            pltpu.SemaphoreType.DMA,
            pltpu.SemaphoreType.DMA,
            pltpu.SemaphoreType.DMA,
            pltpu.SemaphoreType.DMA,
        ],
    )
    def combine(eo_hbm, sh_hbm, p0_hbm, p1_hbm, out_hbm, i0_v, i1_v,
                a0, b0, c0, a1, b1, c1, sa0, sb0, sc0, sa1, sb1, sc1):
        wid = lax.axis_index("s") * 2 + lax.axis_index("c")
        base = wid * _CTOK
        bufs = ((a0, b0, c0, sa0, sb0, sc0), (a1, b1, c1, sa1, sb1, sc1))
        pltpu.sync_copy(p0_hbm.at[wid], i0_v)
        pltpu.sync_copy(p1_hbm.at[wid], i1_v)

        def start(i):
            a, b, c, sa, sb, sc = bufs[i % 2]
            return (pltpu.async_copy(eo_hbm.at[i0_v.at[i]], a, sa),
                    pltpu.async_copy(eo_hbm.at[i1_v.at[i]], b, sb),
                    pltpu.async_copy(sh_hbm.at[pl.ds(base + i * _CCH, _CCH)],
                                     c, sc))

        cps = start(0)
        for i in range(_CN):
            if i + 1 < _CN:
                nxt = start(i + 1)
            for cp in cps:
                cp.wait()
            a, b, c = bufs[i % 2][:3]

            def row(r, _):
                def colstep(j, _):
                    cidx = j * 16
                    c[r, pl.ds(cidx, 16)] = (a[r, pl.ds(cidx, 16)]
                                             + b[r, pl.ds(cidx, 16)]
                                             + c[r, pl.ds(cidx, 16)])
                    return 0

                return lax.fori_loop(0, _NV, colstep, 0)

            lax.fori_loop(0, _CCH, row, 0)
            pltpu.sync_copy(c, out_hbm.at[pl.ds(base + i * _CCH, _CCH)])
            if i + 1 < _CN:
                cps = nxt

    return combine


def _sc_combine(eo, sh, p03, p13):
    return _sc_combine_kernel()(eo, sh, p03, p13)


def kernel(x, Wg, bg, W1, b1, W2, b2, W3, b3, Ws1, bs1, Ws2, bs2, Ws3, bs3):
    xf = x.reshape(N, D)

    wg_pad = jnp.zeros((LANES, D), jnp.float32).at[:E].set(Wg)
    bg_pad = jnp.zeros((1, LANES), jnp.float32).at[0, :E].set(bg)
    w, idx = _route(xf, wg_pad, bg_pad)

    idxf = idx.reshape(A)
    onehot = (idxf[:, None] == jnp.arange(E)[None, :]).astype(jnp.int32)
    ranks = jnp.cumsum(onehot, axis=0) - 1
    rank = jnp.take_along_axis(ranks, idxf[:, None], axis=1)[:, 0]
    counts = onehot.sum(axis=0)
    padded = ((counts + TM - 1) // TM) * TM
    off = jnp.concatenate([jnp.zeros(1, jnp.int32),
                           jnp.cumsum(padded)[:-1].astype(jnp.int32)])
    pos = off[idxf] + rank
    tok = (jnp.arange(A, dtype=jnp.int32) // TOP_K)
    tok_ids = jnp.zeros(CAP, jnp.int32).at[pos].set(tok)
    wsort = jnp.zeros(CAP, jnp.float32).at[pos].set(w.reshape(A))
    ntiles = padded // TM
    cumt = jnp.cumsum(ntiles)
    texp = jnp.sum((jnp.arange(G, dtype=jnp.int32)[:, None]
                    >= cumt[None, :].astype(jnp.int32)).astype(jnp.int32),
                   axis=1)
    valid = (texp < E).astype(jnp.int32)
    texp = jnp.minimum(texp, E - 1)
    posk = pos.reshape(N, TOP_K)
    p0 = posk[:, 0].astype(jnp.int32)
    p1 = posk[:, 1].astype(jnp.int32)

    w1u = _prep_pair(W1)
    w3u = _prep_pair(W3)
    w2b = _prep_cast(W2)
    ws1u = _prep_pair(Ws1[None])
    ws3u = _prep_pair(Ws3[None])
    ws2b = _prep_cast(Ws2[None])

    xs = _sc_gather(xf, tok_ids.reshape(NW, _GN, _GCH))
    sh = _shared_ffn(xf, ws1u, ws3u, ws2b)
    eo = _grouped_ffn(xs, w1u, w3u, w2b,
                      wsort.reshape(G, 1, TM), texp, valid)
    out = _sc_combine(eo, sh, p0.reshape(NW, _CN, _CCH),
                      p1.reshape(NW, _CN, _CCH))
    return out.reshape(x.shape)
